# baseline (speedup 1.0000x reference)
import functools

import jax
import jax.numpy as jnp
from jax.experimental import pallas as pl

_KFRAC = 0.2
_ALPHA = 0.7
_STEPS = 9
_Z = 0.84162123
_D = 0.20


def _pool_body(x_ref, o_ref, *, k_top, n):
    x = x_ref[...]
    rows = x.shape[0]
    k_bot = n - k_top + 1

    def cnt_pair(t_a, t_b):
        comb = jnp.where(x >= t_a, jnp.int32(1), jnp.int32(0)) + jnp.where(
            x >= t_b, jnp.int32(2048), jnp.int32(0)
        )
        cnt = jnp.sum(comb, axis=1, keepdims=True)
        return cnt & jnp.int32(2047), jax.lax.shift_right_logical(cnt, jnp.int32(11))

    mx = jnp.max(x, axis=1, keepdims=True)
    mn = jnp.min(x, axis=1, keepdims=True)
    mu = jnp.mean(x, axis=1, keepdims=True)
    var = jnp.mean(x * x, axis=1, keepdims=True) - mu * mu
    sg = jnp.sqrt(jnp.maximum(var, 0.0))

    lo_a0 = mu + sg * (_Z - _D)
    hi_a0 = mu + sg * (_Z + _D)
    lo_b0 = mu - sg * (_Z + _D)
    hi_b0 = mu - sg * (_Z - _D)

    c_lo_a, c_lo_b = cnt_pair(lo_a0, lo_b0)
    c_hi_a, c_hi_b = cnt_pair(hi_a0, hi_b0)
    lo_a = jnp.where(c_lo_a >= k_top, lo_a0, mn)
    hi_a = jnp.where(c_hi_a < k_top, hi_a0, mx)
    lo_b = jnp.where(c_lo_b >= k_bot, lo_b0, mn)
    hi_b = jnp.where(c_hi_b < k_bot, hi_b0, mx)

    for _ in range(_STEPS):
        mid_a = 0.5 * (lo_a + hi_a)
        mid_b = 0.5 * (lo_b + hi_b)
        c_a, c_b = cnt_pair(mid_a, mid_b)
        ok_a = c_a >= k_top
        ok_b = c_b >= k_bot
        lo_a = jnp.where(ok_a, mid_a, lo_a)
        hi_a = jnp.where(ok_a, hi_a, mid_a)
        lo_b = jnp.where(ok_b, mid_b, lo_b)
        hi_b = jnp.where(ok_b, hi_b, mid_b)

    s_top = k_top * lo_a[:, 0] + jnp.sum(jnp.maximum(x - lo_a, 0.0), axis=1)
    s_bot = k_top * lo_b[:, 0] - jnp.sum(jnp.maximum(lo_b - x, 0.0), axis=1)
    out = (s_top + _ALPHA * s_bot) * (0.5 / k_top)
    o_ref[...] = out.reshape(1, 1, rows)


def kernel(input):
    b, c, h, w = input.shape
    n = h * w
    k_top = int(round(_KFRAC * n))
    rows = b * c
    r_blk = 256
    grid = rows // r_blk
    x = input.reshape(rows, n)

    out = pl.pallas_call(
        functools.partial(_pool_body, k_top=k_top, n=n),
        grid=(grid,),
        in_specs=[pl.BlockSpec((r_blk, n), lambda i: (i, 0))],
        out_specs=pl.BlockSpec((1, 1, r_blk), lambda i: (i, 0, 0)),
        out_shape=jax.ShapeDtypeStruct((grid, 1, r_blk), jnp.float32),
    )(x)
    return out.reshape(b, c)


# speedup vs baseline: 1.3165x; 1.3165x over previous
import functools

import jax
import jax.numpy as jnp
from jax.experimental import pallas as pl
from jax.experimental.pallas import tpu as pltpu

_KFRAC = 0.2
_ALPHA = 0.7
_STEPS = 4
_Z = 0.84162123
_D = 0.20


def _pool_body(x_ref, o_ref, *, k_top, n):
    x = x_ref[...]
    rows = x.shape[0]
    k_bot = n - k_top + 1

    def cnt_pair(t_a, t_b):
        comb = jnp.where(x >= t_a, jnp.int32(1), jnp.int32(0)) + jnp.where(
            x >= t_b, jnp.int32(2048), jnp.int32(0)
        )
        cnt = jnp.sum(comb, axis=1, keepdims=True)
        return cnt & jnp.int32(2047), jax.lax.shift_right_logical(cnt, jnp.int32(11))

    mx = jnp.max(x, axis=1, keepdims=True)
    mn = jnp.min(x, axis=1, keepdims=True)
    mu = jnp.mean(x, axis=1, keepdims=True)
    var = jnp.mean(x * x, axis=1, keepdims=True) - mu * mu
    sg = jnp.sqrt(jnp.maximum(var, 0.0))

    lo_a0 = mu + sg * (_Z - _D)
    hi_a0 = mu + sg * (_Z + _D)
    lo_b0 = mu - sg * (_Z + _D)
    hi_b0 = mu - sg * (_Z - _D)

    c_lo_a, c_lo_b = cnt_pair(lo_a0, lo_b0)
    c_hi_a, c_hi_b = cnt_pair(hi_a0, hi_b0)
    lo_a = jnp.where(c_lo_a >= k_top, lo_a0, mn)
    hi_a = jnp.where(c_hi_a < k_top, hi_a0, mx)
    lo_b = jnp.where(c_lo_b >= k_bot, lo_b0, mn)
    hi_b = jnp.where(c_hi_b < k_bot, hi_b0, mx)

    for _ in range(_STEPS):
        mid_a = 0.5 * (lo_a + hi_a)
        mid_b = 0.5 * (lo_b + hi_b)
        c_a, c_b = cnt_pair(mid_a, mid_b)
        ok_a = c_a >= k_top
        ok_b = c_b >= k_bot
        lo_a = jnp.where(ok_a, mid_a, lo_a)
        hi_a = jnp.where(ok_a, hi_a, mid_a)
        lo_b = jnp.where(ok_b, mid_b, lo_b)
        hi_b = jnp.where(ok_b, hi_b, mid_b)

    s_top = k_top * lo_a[:, 0] + jnp.sum(jnp.maximum(x - lo_a, 0.0), axis=1)
    s_bot = k_top * lo_b[:, 0] - jnp.sum(jnp.maximum(lo_b - x, 0.0), axis=1)
    out = (s_top + _ALPHA * s_bot) * (0.5 / k_top)
    o_ref[...] = out.reshape(1, 1, rows)


def kernel(input):
    b, c, h, w = input.shape
    n = h * w
    k_top = int(round(_KFRAC * n))
    rows = b * c
    r_blk = 512
    grid = rows // r_blk
    x = input.reshape(rows, n)

    out = pl.pallas_call(
        functools.partial(_pool_body, k_top=k_top, n=n),
        grid=(grid,),
        in_specs=[pl.BlockSpec((r_blk, n), lambda i: (i, 0))],
        out_specs=pl.BlockSpec((1, 1, r_blk), lambda i: (i, 0, 0)),
        out_shape=jax.ShapeDtypeStruct((grid, 1, r_blk), jnp.float32),
        compiler_params=pltpu.CompilerParams(
            dimension_semantics=("parallel",)
        ),
    )(x)
    return out.reshape(b, c)
